# baseline (speedup 1.0000x reference)
import functools

import numpy as np
import jax
import jax.numpy as jnp
from jax import lax
from jax.experimental import pallas as pl
from jax.experimental.pallas import tpu as pltpu
from jax.experimental.pallas import tpu_sc as plsc

_B, _T, _E = 32, 1024, 768
_N = _B * _T

_ROT_A = (13, 15, 26, 6)
_ROT_B = (17, 29, 16, 24)


def _host_threefry2x32(k1, k2, x0, x1):
    def rotl(x, d):
        return ((x << np.uint32(d)) | (x >> np.uint32(32 - d))).astype(np.uint32)

    def rounds(x0, x1, rots):
        for r in rots:
            x0 = (x0 + x1).astype(np.uint32)
            x1 = x0 ^ rotl(x1, r)
        return x0, x1

    ks0, ks1 = np.uint32(k1), np.uint32(k2)
    ks2 = np.uint32(ks0 ^ ks1 ^ np.uint32(0x1BD11BDA))
    x0 = (np.uint32(x0) + ks0).astype(np.uint32)
    x1 = (np.uint32(x1) + ks1).astype(np.uint32)
    x0, x1 = rounds(x0, x1, _ROT_A)
    x0 = (x0 + ks1).astype(np.uint32)
    x1 = (x1 + ks2 + np.uint32(1)).astype(np.uint32)
    x0, x1 = rounds(x0, x1, _ROT_B)
    x0 = (x0 + ks2).astype(np.uint32)
    x1 = (x1 + ks0 + np.uint32(2)).astype(np.uint32)
    x0, x1 = rounds(x0, x1, _ROT_A)
    x0 = (x0 + ks0).astype(np.uint32)
    x1 = (x1 + ks1 + np.uint32(3)).astype(np.uint32)
    x0, x1 = rounds(x0, x1, _ROT_B)
    x0 = (x0 + ks1).astype(np.uint32)
    x1 = (x1 + ks2 + np.uint32(4)).astype(np.uint32)
    x0, x1 = rounds(x0, x1, _ROT_A)
    x0 = (x0 + ks2).astype(np.uint32)
    x1 = (x1 + ks0 + np.uint32(5)).astype(np.uint32)
    return x0, x1


_K1, _K2 = (int(v[0]) for v in
            _host_threefry2x32(0, 0, np.uint32([0]), np.uint32([1])))

_DROP_TH = 838861


def _i32c(x):
    return jnp.int32(np.uint32(x & 0xFFFFFFFF).view(np.int32))


def _bits_from_index_i32(idx):
    sru = lax.shift_right_logical

    def rounds(x0, x1, rots):
        for r in rots:
            x0 = x0 + x1
            x1 = x0 ^ ((x1 << jnp.int32(r)) | sru(x1, jnp.int32(32 - r)))
        return x0, x1

    ks0 = _i32c(_K1)
    ks1 = _i32c(_K2)
    ks2 = _i32c(_K1 ^ _K2 ^ 0x1BD11BDA)
    x0 = jnp.full(idx.shape, ks0, jnp.int32)
    x1 = idx + ks1
    x0, x1 = rounds(x0, x1, _ROT_A)
    x0 = x0 + ks1
    x1 = x1 + _i32c((_K1 ^ _K2 ^ 0x1BD11BDA) + 1)
    x0, x1 = rounds(x0, x1, _ROT_B)
    x0 = x0 + ks2
    x1 = x1 + _i32c(_K1 + 2)
    x0, x1 = rounds(x0, x1, _ROT_A)
    x0 = x0 + ks0
    x1 = x1 + _i32c(_K2 + 3)
    x0, x1 = rounds(x0, x1, _ROT_B)
    x0 = x0 + ks1
    x1 = x1 + _i32c((_K1 ^ _K2 ^ 0x1BD11BDA) + 4)
    x0, x1 = rounds(x0, x1, _ROT_A)
    x0 = x0 + ks2
    x1 = x1 + _i32c(_K1 + 5)
    return x0 ^ x1


_NC, _NS = 2, 16
_NW = _NC * _NS
_RPW = _N // _NW
_CB = 16
_NB = _RPW // _CB
_NSLOT = 8


def _sc_body(x_hbm, o_hbm, dlist, flags, zbuf, b0, b1, b2, b3, b4, b5, b6, b7, sg, ss, sz):
    bufs = (b0, b1, b2, b3, b4, b5, b6, b7)
    wid = lax.axis_index("s") * _NC + lax.axis_index("c")
    base = wid * _RPW
    lane = lax.iota(jnp.int32, 16)

    zv = jnp.zeros((16,), jnp.float32)

    @pl.loop(0, 48)
    def _zfill(c):
        for r in range(16):
            zbuf[r, pl.ds(c * 16, 16)] = zv

    def do_group(g, valid):
        rows = base + g * 16 + lane
        bits = _bits_from_index_i32(rows)
        m = lax.shift_right_logical(bits, jnp.int32(9))
        di = lax.shift_right_logical(m - jnp.int32(_DROP_TH), jnp.int32(31))

        @pl.when(valid)
        def _():
            flags[g & 63] = di

    def g_issue(cc):
        for s in range(_NSLOT):
            @pl.when((cc & (_NSLOT - 1)) == s)
            def _(s=s):
                pltpu.async_copy(x_hbm.at[pl.ds(base + cc * _CB, _CB)],
                                 bufs[s], sg)

    def g_wait(cc):
        for s in range(_NSLOT):
            @pl.when((cc & (_NSLOT - 1)) == s)
            def _(s=s):
                pltpu.make_async_copy(x_hbm.at[pl.ds(base, _CB)],
                                      bufs[s], sg).wait()

    def s_issue(cc):
        for s in range(_NSLOT):
            @pl.when((cc & (_NSLOT - 1)) == s)
            def _(s=s):
                pltpu.async_copy(bufs[s],
                                 o_hbm.at[pl.ds(base + cc * _CB, _CB)], ss)

    def s_wait(cc):
        for s in range(_NSLOT):
            @pl.when((cc & (_NSLOT - 1)) == s)
            def _(s=s):
                pltpu.make_async_copy(bufs[s], o_hbm.at[pl.ds(base, _CB)],
                                      ss).wait()

    def master(cc, z):
        valid = cc < _NB

        @pl.when(valid)
        def _():
            @pl.when(cc >= _NSLOT)
            def _():
                s_wait(cc - _NSLOT)
            g_issue(cc)

        do_group(cc & 63, valid)

        @pl.when(jnp.logical_and(cc >= 2, cc - 2 < _NB))
        def _():
            g_wait(cc - 2)
            s_issue(cc - 2)
        return z

    lax.fori_loop(0, _NB + 2, master, jnp.int32(0))

    for j in range(_NSLOT):
        s_wait(jnp.int32(_NB - _NSLOT + j))

    def compact(g, st):
        doff, ld, dP = st
        dv = flags[g]
        for i in range(16):
            fi = dv[i]
            ri = base + g * 16 + i
            d = lane ^ (doff & 15)
            eq = jnp.int32(1) - lax.shift_right_logical(d | (jnp.int32(0) - d),
                                                        jnp.int32(31))
            sel = eq * fi
            dP = dP + sel * (ri - dP)
            ld = ld + fi * (ri - ld)
            doff = doff + fi

            @pl.when(jnp.logical_and(fi > 0, (doff & 15) == 0))
            def _():
                dlist[(doff >> 4) - 1] = dP
        return doff, ld, dP

    doff, ld, dP = lax.fori_loop(
        0, 64, compact,
        (jnp.int32(0), jnp.int32(0), jnp.zeros((16,), jnp.int32)))

    rem = doff & 15

    @pl.when(rem != 0)
    def _():
        lt = lax.shift_right_logical(lane - rem, jnp.int32(31))
        dlist[doff >> 4] = ld + lt * (dP - ld)

    nbd = (doff + 15) >> 4

    @pl.loop(0, nbd)
    def _zscat(j):
        pltpu.async_copy(zbuf, o_hbm.at[dlist.at[j]], sz)

    @pl.loop(0, nbd)
    def _zdrain(j):
        pltpu.make_async_copy(zbuf, o_hbm.at[dlist.at[0]], sz).wait()


_sc_call = functools.partial(
    pl.kernel,
    out_type=jax.ShapeDtypeStruct((_N, _E), jnp.float32),
    mesh=plsc.VectorSubcoreMesh(core_axis_name="c", subcore_axis_name="s"),
    scratch_types=[
        pltpu.VMEM((_RPW // 16, 16), jnp.int32),
        pltpu.VMEM((64, 16), jnp.int32),
        pltpu.VMEM((16, _E), jnp.float32),
        pltpu.VMEM((_CB, _E), jnp.float32),
        pltpu.VMEM((_CB, _E), jnp.float32),
        pltpu.VMEM((_CB, _E), jnp.float32),
        pltpu.VMEM((_CB, _E), jnp.float32),
        pltpu.VMEM((_CB, _E), jnp.float32),
        pltpu.VMEM((_CB, _E), jnp.float32),
        pltpu.VMEM((_CB, _E), jnp.float32),
        pltpu.VMEM((_CB, _E), jnp.float32),
        pltpu.SemaphoreType.DMA,
        pltpu.SemaphoreType.DMA,
        pltpu.SemaphoreType.DMA,
    ],
)(_sc_body)


def kernel(X):
    Xf = X.reshape(_N, _E)
    out = _sc_call(Xf)
    return out.reshape(_B, _T, _E)


# speedup vs baseline: 1.6026x; 1.6026x over previous
import numpy as np
import jax
import jax.numpy as jnp
from jax.experimental import pallas as pl

_P = 0.1
_B, _T, _E = 32, 1024, 768
_N = _B * _T

_ROT_A = (13, 15, 26, 6)
_ROT_B = (17, 29, 16, 24)


def _host_threefry2x32(k1, k2, x0, x1):
    def rotl(x, d):
        return ((x << np.uint32(d)) | (x >> np.uint32(32 - d))).astype(np.uint32)

    def rounds(x0, x1, rots):
        for r in rots:
            x0 = (x0 + x1).astype(np.uint32)
            x1 = x0 ^ rotl(x1, r)
        return x0, x1

    ks0, ks1 = np.uint32(k1), np.uint32(k2)
    ks2 = np.uint32(ks0 ^ ks1 ^ np.uint32(0x1BD11BDA))
    x0 = (np.uint32(x0) + ks0).astype(np.uint32)
    x1 = (np.uint32(x1) + ks1).astype(np.uint32)
    x0, x1 = rounds(x0, x1, _ROT_A)
    x0 = (x0 + ks1).astype(np.uint32)
    x1 = (x1 + ks2 + np.uint32(1)).astype(np.uint32)
    x0, x1 = rounds(x0, x1, _ROT_B)
    x0 = (x0 + ks2).astype(np.uint32)
    x1 = (x1 + ks0 + np.uint32(2)).astype(np.uint32)
    x0, x1 = rounds(x0, x1, _ROT_A)
    x0 = (x0 + ks0).astype(np.uint32)
    x1 = (x1 + ks1 + np.uint32(3)).astype(np.uint32)
    x0, x1 = rounds(x0, x1, _ROT_B)
    x0 = (x0 + ks1).astype(np.uint32)
    x1 = (x1 + ks2 + np.uint32(4)).astype(np.uint32)
    x0, x1 = rounds(x0, x1, _ROT_A)
    x0 = (x0 + ks2).astype(np.uint32)
    x1 = (x1 + ks0 + np.uint32(5)).astype(np.uint32)
    return x0, x1


_K1, _K2 = (int(v[0]) for v in
            _host_threefry2x32(0, 0, np.uint32([0]), np.uint32([1])))


def _bits_from_index(idx):
    sru = jax.lax.shift_right_logical

    def rounds(x0, x1, rots):
        for r in rots:
            x0 = x0 + x1
            x1 = x0 ^ ((x1 << jnp.uint32(r)) | sru(x1, jnp.uint32(32 - r)))
        return x0, x1

    ks0 = jnp.uint32(_K1)
    ks1 = jnp.uint32(_K2)
    ks2 = jnp.uint32(_K1 ^ _K2 ^ 0x1BD11BDA)
    x0 = jnp.full(idx.shape, ks0, jnp.uint32)
    x1 = idx + ks1
    x0, x1 = rounds(x0, x1, _ROT_A)
    x0 = x0 + ks1
    x1 = x1 + (ks2 + jnp.uint32(1))
    x0, x1 = rounds(x0, x1, _ROT_B)
    x0 = x0 + ks2
    x1 = x1 + (ks0 + jnp.uint32(2))
    x0, x1 = rounds(x0, x1, _ROT_A)
    x0 = x0 + ks0
    x1 = x1 + (ks1 + jnp.uint32(3))
    x0, x1 = rounds(x0, x1, _ROT_B)
    x0 = x0 + ks1
    x1 = x1 + (ks2 + jnp.uint32(4))
    x0, x1 = rounds(x0, x1, _ROT_A)
    x0 = x0 + ks2
    x1 = x1 + (ks0 + jnp.uint32(5))
    return x0 ^ x1


def _keep_from_index(idx):
    bits = _bits_from_index(idx)
    fb = jax.lax.shift_right_logical(bits, jnp.uint32(9)) | jnp.uint32(0x3F800000)
    u = jax.lax.bitcast_convert_type(fb, jnp.float32) - jnp.float32(1.0)
    return jnp.where(u < jnp.float32(_P), jnp.float32(0.0), jnp.float32(1.0))


_MROWS, _MCOLS = 256, 128


def _mask_kernel(o_ref):
    s = jax.lax.broadcasted_iota(jnp.uint32, (_MROWS, _MCOLS), 0)
    l = jax.lax.broadcasted_iota(jnp.uint32, (_MROWS, _MCOLS), 1)
    o_ref[...] = _keep_from_index(s * jnp.uint32(_MCOLS) + l)


_BLK = 4096


def _fused_kernel(x_ref, o_ref, m_ref):
    i = pl.program_id(0)

    @pl.when(i == 0)
    def _():
        s = jax.lax.broadcasted_iota(jnp.uint32, (_MROWS, _MCOLS), 0)
        l = jax.lax.broadcasted_iota(jnp.uint32, (_MROWS, _MCOLS), 1)
        m_ref[...] = _keep_from_index(s * jnp.uint32(_MCOLS) + l)

    rows_per = _BLK // _MCOLS
    m = m_ref[pl.ds(i * rows_per, rows_per), :]
    b = jnp.reshape(
        jax.lax.broadcast_in_dim(m, (rows_per, _MCOLS, _MCOLS), (0, 2)),
        (_BLK, _MCOLS))
    r_sub = jax.lax.broadcasted_iota(jnp.int32, (_BLK, _MCOLS), 0)
    r_lane = jax.lax.broadcasted_iota(jnp.int32, (_BLK, _MCOLS), 1)
    onehot = (r_sub % _MCOLS == r_lane).astype(jnp.float32)
    mcol = jnp.sum(b * onehot, axis=1, keepdims=True)
    o_ref[...] = x_ref[...] * mcol


def kernel(X):
    from jax.experimental.pallas import tpu as pltpu
    Xf = X.reshape(_N, _E)
    out = pl.pallas_call(
        _fused_kernel,
        grid=(_N // _BLK,),
        in_specs=[
            pl.BlockSpec((_BLK, _E), lambda i: (i, 0)),
        ],
        out_specs=pl.BlockSpec((_BLK, _E), lambda i: (i, 0)),
        out_shape=jax.ShapeDtypeStruct((_N, _E), jnp.float32),
        scratch_shapes=[pltpu.VMEM((_MROWS, _MCOLS), jnp.float32)],
    )(Xf)
    return out.reshape(_B, _T, _E)
